# baseline (speedup 1.0000x reference)
import functools

import jax
import jax.numpy as jnp
from jax import lax
from jax.experimental import pallas as pl
from jax.experimental.pallas import tpu as pltpu
from jax.experimental.pallas import tpu_sc as plsc

N, C, H, W = 8, 96, 112, 112
OH, OW = 2 * H, 2 * W
PLANES = N * C
NUM_WORKERS = 32
PLANES_PER_WORKER = PLANES // NUM_WORKERS
VECS_PER_ROW = W // 16


def _unpool_body(x_hbm, idx_hbm, out_hbm, x_v, idx_v, plane_v):
    cid = lax.axis_index("c")
    sid = lax.axis_index("s")
    wid = sid * 2 + cid

    zeros16 = jnp.zeros((16,), jnp.float32)

    def zero_body(r, carry):
        for k in range(OW // 16):
            plane_v[r, pl.ds(k * 16, 16)] = zeros16
        return carry

    lax.fori_loop(0, OH, zero_body, 0)

    def plane_body(t, carry):
        p = wid * PLANES_PER_WORKER + t
        n = p // C
        c = p - n * C
        pltpu.sync_copy(x_hbm.at[n, c], x_v)
        pltpu.sync_copy(idx_hbm.at[n, c], idx_v)

        def scat(r, carry):
            for k in range(VECS_PER_ROW):
                iv = idx_v[r, pl.ds(k * 16, 16)]
                xv = x_v[r, pl.ds(k * 16, 16)]
                rr = iv // OW
                cc = iv - rr * OW
                plsc.store_scatter(plane_v, [rr, cc], xv)
            return carry

        lax.fori_loop(0, H, scat, 0)
        pltpu.sync_copy(plane_v, out_hbm.at[n, c])

        def unscat(r, carry):
            for k in range(VECS_PER_ROW):
                iv = idx_v[r, pl.ds(k * 16, 16)]
                rr = iv // OW
                cc = iv - rr * OW
                plsc.store_scatter(plane_v, [rr, cc], zeros16)
            return carry

        lax.fori_loop(0, H, unscat, 0)
        return carry

    lax.fori_loop(0, PLANES_PER_WORKER, plane_body, 0)


@jax.jit
def _unpool(x, idx):
    mesh = plsc.VectorSubcoreMesh(core_axis_name="c", subcore_axis_name="s")
    f = functools.partial(
        pl.kernel,
        out_type=jax.ShapeDtypeStruct((N, C, OH, OW), jnp.float32),
        mesh=mesh,
        scratch_types=[
            pltpu.VMEM((H, W), jnp.float32),
            pltpu.VMEM((H, W), jnp.int32),
            pltpu.VMEM((OH, OW), jnp.float32),
        ],
        compiler_params=pltpu.CompilerParams(needs_layout_passes=False),
    )(_unpool_body)
    return f(x, idx)


def kernel(x, indices):
    return _unpool(x, indices.astype(jnp.int32))


# speedup vs baseline: 3.4664x; 3.4664x over previous
import functools

import jax
import jax.numpy as jnp
from jax import lax
from jax.experimental import pallas as pl
from jax.experimental.pallas import tpu as pltpu
from jax.experimental.pallas import tpu_sc as plsc

N, C, H, W = 8, 96, 112, 112
OH, OW = 2 * H, 2 * W
PLANES = N * C
NUM_WORKERS = 32
PLANES_PER_WORKER = PLANES // NUM_WORKERS
VECS_PER_ROW = W // 16


def _unpool_body(x_hbm, idx_hbm, out_hbm, x_v, idx_v, plane_v):
    cid = lax.axis_index("c")
    sid = lax.axis_index("s")
    wid = sid * 2 + cid

    zeros16 = jnp.zeros((16,), jnp.float32)

    def zero_body(r, carry):
        for k in range(OW // 16):
            plane_v[r, pl.ds(k * 16, 16)] = zeros16
        return carry

    lax.fori_loop(0, OH, zero_body, 0)

    def plane_body(t, carry):
        p = wid * PLANES_PER_WORKER + t
        n = p // C
        c = p - n * C
        pltpu.sync_copy(x_hbm.at[n, c], x_v)
        pltpu.sync_copy(idx_hbm.at[n, c], idx_v)

        def scat(r, carry):
            base = 448 * r
            for k in range(VECS_PER_ROW):
                iv = idx_v[r, pl.ds(k * 16, 16)]
                xv = x_v[r, pl.ds(k * 16, 16)]
                iv2 = iv - base
                b = (iv2 >= OW).astype(jnp.int32)
                rr = 2 * r + b
                cc = iv2 - OW * b
                plsc.store_scatter(plane_v, [rr, cc], xv)
            return carry

        lax.fori_loop(0, H, scat, 0)
        pltpu.sync_copy(plane_v, out_hbm.at[n, c])

        def unscat(r, carry):
            base = 448 * r
            for k in range(VECS_PER_ROW):
                iv = idx_v[r, pl.ds(k * 16, 16)]
                iv2 = iv - base
                b = (iv2 >= OW).astype(jnp.int32)
                rr = 2 * r + b
                cc = iv2 - OW * b
                plsc.store_scatter(plane_v, [rr, cc], zeros16)
            return carry

        lax.fori_loop(0, H, unscat, 0)
        return carry

    lax.fori_loop(0, PLANES_PER_WORKER, plane_body, 0)


@jax.jit
def _unpool(x, idx):
    mesh = plsc.VectorSubcoreMesh(core_axis_name="c", subcore_axis_name="s")
    f = functools.partial(
        pl.kernel,
        out_type=jax.ShapeDtypeStruct((N, C, OH, OW), jnp.float32),
        mesh=mesh,
        scratch_types=[
            pltpu.VMEM((H, W), jnp.float32),
            pltpu.VMEM((H, W), jnp.int32),
            pltpu.VMEM((OH, OW), jnp.float32),
        ],
        compiler_params=pltpu.CompilerParams(needs_layout_passes=False),
    )(_unpool_body)
    return f(x, idx)


def kernel(x, indices):
    return _unpool(x, indices.astype(jnp.int32))


# speedup vs baseline: 4.8839x; 1.4090x over previous
import functools

import jax
import jax.numpy as jnp
from jax import lax
from jax.experimental import pallas as pl
from jax.experimental.pallas import tpu as pltpu
from jax.experimental.pallas import tpu_sc as plsc

N, C, H, W = 8, 96, 112, 112
OH, OW = 2 * H, 2 * W
PLANES = N * C
NUM_WORKERS = 32
PLANES_PER_WORKER = PLANES // NUM_WORKERS
VECS_PER_ROW = W // 16


def _unpool_body(x_hbm, idx_hbm, out_hbm, x_v, idx_v, plane_v):
    cid = lax.axis_index("c")
    sid = lax.axis_index("s")
    wid = sid * 2 + cid

    zeros16 = jnp.zeros((16,), jnp.float32)

    def plane_body(t, carry):
        p = wid * PLANES_PER_WORKER + t
        n = p // C
        c = p - n * C
        pltpu.sync_copy(x_hbm.at[n, c], x_v)
        pltpu.sync_copy(idx_hbm.at[n, c], idx_v)

        def zero_body(r2, carry):
            for k in range(OW // 16):
                plane_v[r2, pl.ds(k * 16, 16)] = zeros16
            return carry

        lax.fori_loop(0, OH, zero_body, 0, unroll=2)

        def scat(r, carry):
            base = 448 * r
            for k in range(VECS_PER_ROW):
                iv = idx_v[r, pl.ds(k * 16, 16)]
                xv = x_v[r, pl.ds(k * 16, 16)]
                iv2 = iv - base
                b = (iv2 >= OW).astype(jnp.int32)
                rr = 2 * r + b
                cc = iv2 - OW * b
                plsc.store_scatter(plane_v, [rr, cc], xv)
            return carry

        lax.fori_loop(0, H, scat, 0, unroll=2)
        pltpu.sync_copy(plane_v, out_hbm.at[n, c])
        return carry

    lax.fori_loop(0, PLANES_PER_WORKER, plane_body, 0)


@jax.jit
def _unpool(x, idx):
    mesh = plsc.VectorSubcoreMesh(core_axis_name="c", subcore_axis_name="s")
    f = functools.partial(
        pl.kernel,
        out_type=jax.ShapeDtypeStruct((N, C, OH, OW), jnp.float32),
        mesh=mesh,
        scratch_types=[
            pltpu.VMEM((H, W), jnp.float32),
            pltpu.VMEM((H, W), jnp.int32),
            pltpu.VMEM((OH, OW), jnp.float32),
        ],
        compiler_params=pltpu.CompilerParams(needs_layout_passes=False),
    )(_unpool_body)
    return f(x, idx)


def kernel(x, indices):
    return _unpool(x, indices.astype(jnp.int32))


# speedup vs baseline: 6.8938x; 1.4115x over previous
import functools

import jax
import jax.numpy as jnp
from jax import lax
from jax.experimental import pallas as pl
from jax.experimental.pallas import tpu as pltpu
from jax.experimental.pallas import tpu_sc as plsc

N, C, H, W = 8, 96, 112, 112
OH, OW = 2 * H, 2 * W
PLANES = N * C
NUM_WORKERS = 32
PLANES_PER_WORKER = PLANES // NUM_WORKERS
HH = H // 2
OHH = OH // 2
UNITS = 2 * PLANES_PER_WORKER
VECS_PER_ROW = W // 16


def _unpool_body(x_hbm, idx_hbm, out_hbm,
                 x0, x1, i0, i1, p0, p1,
                 si0, si1, so0, so1):
    cid = lax.axis_index("c")
    sid = lax.axis_index("s")
    wid = sid * 2 + cid
    plane0 = wid * PLANES_PER_WORKER

    xbuf = (x0, x1)
    ibuf = (i0, i1)
    pbuf = (p0, p1)
    sin = (si0, si1)
    sout = (so0, so1)

    zeros16 = jnp.zeros((16,), jnp.float32)

    def unit_refs(u):
        t = u // 2
        h = u - 2 * t
        p = plane0 + t
        n = p // C
        c = p - n * C
        xsrc = x_hbm.at[n, c, pl.ds(h * HH, HH)]
        isrc = idx_hbm.at[n, c, pl.ds(h * HH, HH)]
        odst = out_hbm.at[n, c, pl.ds(h * OHH, OHH)]
        return xsrc, isrc, odst, h

    def start_in(u, par):
        xsrc, isrc, _, _ = unit_refs(u)
        pltpu.async_copy(xsrc, xbuf[par], sin[par])
        pltpu.async_copy(isrc, ibuf[par], sin[par])

    def wait_in(u, par):
        xsrc, isrc, _, _ = unit_refs(u)
        pltpu.make_async_copy(xsrc, xbuf[par], sin[par]).wait()
        pltpu.make_async_copy(isrc, ibuf[par], sin[par]).wait()

    def start_out(u, par):
        _, _, odst, _ = unit_refs(u)
        pltpu.async_copy(pbuf[par], odst, sout[par])

    def wait_out(u, par):
        _, _, odst, _ = unit_refs(u)
        pltpu.make_async_copy(pbuf[par], odst, sout[par]).wait()

    def compute(u, par):
        x_v, idx_v, plane_v = xbuf[par], ibuf[par], pbuf[par]
        _, _, _, h = unit_refs(u)

        def zero_body(r2, carry):
            for k in range(OW // 16):
                plane_v[r2, pl.ds(k * 16, 16)] = zeros16
            return carry

        lax.fori_loop(0, OHH, zero_body, 0, unroll=4)

        def scat(rl, carry):
            base = 448 * (h * HH + rl)
            for k in range(VECS_PER_ROW):
                iv = idx_v[rl, pl.ds(k * 16, 16)]
                xv = x_v[rl, pl.ds(k * 16, 16)]
                iv2 = iv - base
                b = (iv2 >= OW).astype(jnp.int32)
                rr = 2 * rl + b
                cc = iv2 - OW * b
                plsc.store_scatter(plane_v, [rr, cc], xv)
            return carry

        lax.fori_loop(0, HH, scat, 0, unroll=2)

    start_in(0, 0)
    start_in(1, 1)

    def step(tt, carry):
        for hpar in (0, 1):
            u = 2 * tt + hpar
            wait_in(u, hpar)

            @pl.when(tt >= 1)
            def _():
                wait_out(u - 2, hpar)

            compute(u, hpar)
            start_out(u, hpar)

            @pl.when(tt < PLANES_PER_WORKER - 1)
            def _():
                start_in(u + 2, hpar)

        return carry

    lax.fori_loop(0, PLANES_PER_WORKER, step, 0)
    wait_out(UNITS - 2, 0)
    wait_out(UNITS - 1, 1)


@jax.jit
def _unpool(x, idx):
    mesh = plsc.VectorSubcoreMesh(core_axis_name="c", subcore_axis_name="s")
    f = functools.partial(
        pl.kernel,
        out_type=jax.ShapeDtypeStruct((N, C, OH, OW), jnp.float32),
        mesh=mesh,
        scratch_types=[
            pltpu.VMEM((HH, W), jnp.float32),
            pltpu.VMEM((HH, W), jnp.float32),
            pltpu.VMEM((HH, W), jnp.int32),
            pltpu.VMEM((HH, W), jnp.int32),
            pltpu.VMEM((OHH, OW), jnp.float32),
            pltpu.VMEM((OHH, OW), jnp.float32),
            pltpu.SemaphoreType.DMA,
            pltpu.SemaphoreType.DMA,
            pltpu.SemaphoreType.DMA,
            pltpu.SemaphoreType.DMA,
        ],
        compiler_params=pltpu.CompilerParams(needs_layout_passes=False),
    )(_unpool_body)
    return f(x, idx)


def kernel(x, indices):
    return _unpool(x, indices.astype(jnp.int32))


# speedup vs baseline: 13.7207x; 1.9903x over previous
import functools

import jax
import jax.numpy as jnp
from jax import lax
from jax.experimental import pallas as pl
from jax.experimental.pallas import tpu as pltpu
from jax.experimental.pallas import tpu_sc as plsc

N, C, H, W = 8, 96, 112, 112
OH, OW = 2 * H, 2 * W
PLANES = N * C
NUM_WORKERS = 32
PLANES_PER_WORKER = PLANES // NUM_WORKERS
HH = H // 2
OHH = OH // 2
UNITS = 2 * PLANES_PER_WORKER
VECS_PER_ROW = W // 16


def _unpool_body(x_hbm, idx_hbm, out_hbm,
                 x0, x1, i0, i1, p0, p1,
                 si0, si1, so0, so1):
    cid = lax.axis_index("c")
    sid = lax.axis_index("s")
    wid = sid * 2 + cid
    plane0 = wid * PLANES_PER_WORKER

    xbuf = (x0, x1)
    ibuf = (i0, i1)
    pbuf = (p0, p1)
    sin = (si0, si1)
    sout = (so0, so1)

    zeros16 = jnp.zeros((16,), jnp.float32)

    def unit_refs(u):
        t = u // 2
        h = u - 2 * t
        p = plane0 + t
        n = p // C
        c = p - n * C
        xsrc = x_hbm.at[n, c, pl.ds(h * HH, HH)]
        isrc = idx_hbm.at[n, c, pl.ds(h * HH, HH)]
        odst = out_hbm.at[n, c, pl.ds(h * OHH, OHH)]
        return xsrc, isrc, odst, h

    def start_in(u, par):
        xsrc, isrc, _, _ = unit_refs(u)
        pltpu.async_copy(xsrc, xbuf[par], sin[par])
        pltpu.async_copy(isrc, ibuf[par], sin[par])

    def wait_in(u, par):
        xsrc, isrc, _, _ = unit_refs(u)
        pltpu.make_async_copy(xsrc, xbuf[par], sin[par]).wait()
        pltpu.make_async_copy(isrc, ibuf[par], sin[par]).wait()

    def start_out(u, par):
        _, _, odst, _ = unit_refs(u)
        pltpu.async_copy(pbuf[par], odst, sout[par])

    def wait_out(u, par):
        _, _, odst, _ = unit_refs(u)
        pltpu.make_async_copy(pbuf[par], odst, sout[par]).wait()

    def compute(u, par):
        x_v, idx_v, plane_v = xbuf[par], ibuf[par], pbuf[par]
        _, _, _, h = unit_refs(u)

        def zero_body(r2, carry):
            for k in range(OW // 16):
                plane_v[r2, pl.ds(k * 16, 16)] = zeros16
            return carry

        lax.fori_loop(0, OHH, zero_body, 0, unroll=4)

        def scat(rl, carry):
            base = 448 * (h * HH + rl)
            ivs = [idx_v[rl, pl.ds(k * 16, 16)] for k in range(VECS_PER_ROW)]
            xvs = [x_v[rl, pl.ds(k * 16, 16)] for k in range(VECS_PER_ROW)]
            addrs = []
            for k in range(VECS_PER_ROW):
                iv2 = ivs[k] - base
                b = (iv2 >= OW).astype(jnp.int32)
                rr = 2 * rl + b
                cc = iv2 - OW * b
                addrs.append((rr, cc))
            for k in range(VECS_PER_ROW):
                rr, cc = addrs[k]
                plsc.store_scatter(plane_v, [rr, cc], xvs[k])
            return carry

        lax.fori_loop(0, HH, scat, 0, unroll=2)

    start_in(0, 0)
    start_in(1, 1)

    def step(tt, carry):
        for hpar in (0, 1):
            u = 2 * tt + hpar
            wait_in(u, hpar)

            @pl.when(tt >= 1)
            def _():
                wait_out(u - 2, hpar)

            compute(u, hpar)
            start_out(u, hpar)

            @pl.when(tt < PLANES_PER_WORKER - 1)
            def _():
                start_in(u + 2, hpar)

        return carry

    lax.fori_loop(0, PLANES_PER_WORKER, step, 0)
    wait_out(UNITS - 2, 0)
    wait_out(UNITS - 1, 1)


@jax.jit
def _unpool(x, idx):
    mesh = plsc.VectorSubcoreMesh(core_axis_name="c", subcore_axis_name="s")
    f = functools.partial(
        pl.kernel,
        out_type=jax.ShapeDtypeStruct((N, C, OH, OW), jnp.float32),
        mesh=mesh,
        scratch_types=[
            pltpu.VMEM((HH, W), jnp.float32),
            pltpu.VMEM((HH, W), jnp.float32),
            pltpu.VMEM((HH, W), jnp.int32),
            pltpu.VMEM((HH, W), jnp.int32),
            pltpu.VMEM((OHH, OW), jnp.float32),
            pltpu.VMEM((OHH, OW), jnp.float32),
            pltpu.SemaphoreType.DMA,
            pltpu.SemaphoreType.DMA,
            pltpu.SemaphoreType.DMA,
            pltpu.SemaphoreType.DMA,
        ],
        compiler_params=pltpu.CompilerParams(needs_layout_passes=False),
    )(_unpool_body)
    return f(x, idx)


def kernel(x, indices):
    return _unpool(x, indices.astype(jnp.int32))


# speedup vs baseline: 13.7465x; 1.0019x over previous
import functools

import jax
import jax.numpy as jnp
from jax import lax
from jax.experimental import pallas as pl
from jax.experimental.pallas import tpu as pltpu
from jax.experimental.pallas import tpu_sc as plsc

N, C, H, W = 8, 96, 112, 112
OH, OW = 2 * H, 2 * W
PLANES = N * C
NUM_WORKERS = 32
PLANES_PER_WORKER = PLANES // NUM_WORKERS
HH = H // 2
OHH = OH // 2
UNITS = 2 * PLANES_PER_WORKER
VECS_PER_ROW = W // 16


def _unpool_body(x_hbm, idx_hbm, out_hbm,
                 x0, x1, i0, i1, p0, p1,
                 si0, si1, so0, so1):
    cid = lax.axis_index("c")
    sid = lax.axis_index("s")
    wid = sid * 2 + cid
    plane0 = wid * PLANES_PER_WORKER

    xbuf = (x0, x1)
    ibuf = (i0, i1)
    pbuf = (p0, p1)
    sin = (si0, si1)
    sout = (so0, so1)

    zeros16 = jnp.zeros((16,), jnp.float32)

    def unit_refs(u):
        t = u // 2
        h = u - 2 * t
        p = plane0 + t
        n = p // C
        c = p - n * C
        xsrc = x_hbm.at[n, c, pl.ds(h * HH, HH)]
        isrc = idx_hbm.at[n, c, pl.ds(h * HH, HH)]
        odst = out_hbm.at[n, c, pl.ds(h * OHH, OHH)]
        return xsrc, isrc, odst, h

    def start_in(u, par):
        xsrc, isrc, _, _ = unit_refs(u)
        pltpu.async_copy(xsrc, xbuf[par], sin[par])
        pltpu.async_copy(isrc, ibuf[par], sin[par])

    def wait_in(u, par):
        xsrc, isrc, _, _ = unit_refs(u)
        pltpu.make_async_copy(xsrc, xbuf[par], sin[par]).wait()
        pltpu.make_async_copy(isrc, ibuf[par], sin[par]).wait()

    def start_out(u, par):
        _, _, odst, _ = unit_refs(u)
        pltpu.async_copy(pbuf[par], odst, sout[par])

    def wait_out(u, par):
        _, _, odst, _ = unit_refs(u)
        pltpu.make_async_copy(pbuf[par], odst, sout[par]).wait()

    def compute(u, par):
        x_v, idx_v, plane_v = xbuf[par], ibuf[par], pbuf[par]
        _, _, _, h = unit_refs(u)

        def zero_body(r2, carry):
            for k in range(OW // 16):
                plane_v[r2, pl.ds(k * 16, 16)] = zeros16
            return carry

        lax.fori_loop(0, OHH, zero_body, 0, unroll=8)

        def scat(rl, carry):
            base = 448 * (h * HH + rl)
            ivs = [idx_v[rl, pl.ds(k * 16, 16)] for k in range(VECS_PER_ROW)]
            xvs = [x_v[rl, pl.ds(k * 16, 16)] for k in range(VECS_PER_ROW)]
            addrs = []
            for k in range(VECS_PER_ROW):
                iv2 = ivs[k] - base
                b = (iv2 >= OW).astype(jnp.int32)
                rr = 2 * rl + b
                cc = iv2 - OW * b
                addrs.append((rr, cc))
            for k in range(VECS_PER_ROW):
                rr, cc = addrs[k]
                plsc.store_scatter(plane_v, [rr, cc], xvs[k])
            return carry

        lax.fori_loop(0, HH, scat, 0, unroll=2)

    start_in(0, 0)
    start_in(1, 1)

    def step(tt, carry):
        for hpar in (0, 1):
            u = 2 * tt + hpar
            wait_in(u, hpar)

            @pl.when(tt >= 1)
            def _():
                wait_out(u - 2, hpar)

            compute(u, hpar)
            start_out(u, hpar)

            @pl.when(tt < PLANES_PER_WORKER - 1)
            def _():
                start_in(u + 2, hpar)

        return carry

    lax.fori_loop(0, PLANES_PER_WORKER, step, 0)
    wait_out(UNITS - 2, 0)
    wait_out(UNITS - 1, 1)


@jax.jit
def _unpool(x, idx):
    mesh = plsc.VectorSubcoreMesh(core_axis_name="c", subcore_axis_name="s")
    f = functools.partial(
        pl.kernel,
        out_type=jax.ShapeDtypeStruct((N, C, OH, OW), jnp.float32),
        mesh=mesh,
        scratch_types=[
            pltpu.VMEM((HH, W), jnp.float32),
            pltpu.VMEM((HH, W), jnp.float32),
            pltpu.VMEM((HH, W), jnp.int32),
            pltpu.VMEM((HH, W), jnp.int32),
            pltpu.VMEM((OHH, OW), jnp.float32),
            pltpu.VMEM((OHH, OW), jnp.float32),
            pltpu.SemaphoreType.DMA,
            pltpu.SemaphoreType.DMA,
            pltpu.SemaphoreType.DMA,
            pltpu.SemaphoreType.DMA,
        ],
        compiler_params=pltpu.CompilerParams(needs_layout_passes=False),
    )(_unpool_body)
    return f(x, idx)


def kernel(x, indices):
    return _unpool(x, indices.astype(jnp.int32))
